# baseline (speedup 1.0000x reference)
import functools

import jax
import jax.numpy as jnp
from jax import lax
from jax.experimental import pallas as pl
from jax.experimental.pallas import tpu as pltpu
from jax.experimental.pallas import tpu_sc as plsc

LANES = 16
KPAD = 32
BI = 16384


def _make_relayout(V, E):
    NB = pl.cdiv(V, 2 * BI)
    H = NB * BI
    NBV = pl.cdiv(V, BI) - 1

    def body(up_ref, lo_ref, dst_ref):
        t_up = jnp.transpose(up_ref[...])
        t_lo = jnp.transpose(lo_ref[...])
        dst_ref[...] = jnp.concatenate([t_up, t_lo], axis=1)

    call = pl.pallas_call(
        body,
        grid=(NB,),
        in_specs=[
            pl.BlockSpec((E, BI), lambda i: (0, i)),
            pl.BlockSpec((E, BI), lambda i: (0, jnp.minimum(i + NB, NBV))),
        ],
        out_specs=pl.BlockSpec((BI, 2 * E), lambda i: (i, 0)),
        out_shape=jax.ShapeDtypeStruct((H, 2 * E), jnp.float32),
    )

    def relayout(table):
        tT = table.T
        return call(tT, tT).reshape(2 * H, E)

    return relayout, H


def _make_sc_relayout(V, E, H):
    info = plsc.get_sparse_core_info()
    NW = info.num_cores * info.num_subcores
    W = H // 128
    w_per_worker = W // NW
    SUB = E // 8
    WC_MAX = (pl.cdiv(V, 128) - 1) * 128

    mesh = plsc.VectorSubcoreMesh(core_axis_name="c", subcore_axis_name="s")

    @functools.partial(
        pl.kernel,
        out_type=jax.ShapeDtypeStruct((2 * H * E,), jnp.float32),
        mesh=mesh,
        compiler_params=pltpu.CompilerParams(needs_layout_passes=False,
                                             use_tc_tiling_on_sc=True),
        scratch_types=[
            pltpu.VMEM((8, SUB, 128), jnp.float32),
            pltpu.VMEM((8, SUB, 128), jnp.float32),
            pltpu.VMEM((256 * E,), jnp.float32),
            pltpu.SemaphoreType.DMA,
        ],
    )
    def sc_relayout(src_hbm, out_hbm, win_up, win_lo, obuf, sem):
        wid = lax.axis_index("s") * info.num_cores + lax.axis_index("c")
        lane = lax.iota(jnp.int32, LANES)
        a_idx = [(16 * v + lane) // 8 for v in range(E // LANES)]
        s_idx = [(16 * v + lane) % 8 for v in range(E // LANES)]

        def window_body(wl, _):
            w = wid * w_per_worker + wl
            c0 = 128 * w
            wc = jnp.minimum(H + c0, WC_MAX)
            copies = []
            for a in range(8):
                copies.append(pltpu.async_copy(
                    src_hbm.at[a, :, pl.ds(c0, 128)], win_up.at[a], sem))
                copies.append(pltpu.async_copy(
                    src_hbm.at[a, :, pl.ds(wc, 128)], win_lo.at[a], sem))
            for cp in copies:
                cp.wait()

            def lid_body(lid, _):
                lsp = jnp.full((LANES,), 0, jnp.int32) + lid
                for half, win in ((0, win_up), (1, win_lo)):
                    rbase = (2 * lid + half) * E
                    for v in range(E // LANES):
                        x = plsc.load_gather(win, [a_idx[v], s_idx[v], lsp])
                        obuf[pl.ds(rbase + v * LANES, LANES)] = x
                return 0

            lax.fori_loop(0, 128, lid_body, 0)
            pltpu.sync_copy(obuf, out_hbm.at[pl.ds(2 * c0 * E, 256 * E)])
            return 0

        lax.fori_loop(0, w_per_worker, window_body, 0)

    return sc_relayout


def _make_sc_dots(B, K, E, H):
    info = plsc.get_sparse_core_info()
    NW = info.num_cores * info.num_subcores
    rows_per_w = B // NW
    C = 32
    n_pairs = rows_per_w // (2 * C)
    EV = E // LANES
    IDX_BLK = 128

    mesh = plsc.VectorSubcoreMesh(core_axis_name="c", subcore_axis_name="s")

    def one_set():
        return [
            pltpu.VMEM((C,), jnp.int32),
            pltpu.VMEM((C,), jnp.int32),
            pltpu.VMEM((C * K,), jnp.int32),
            pltpu.VMEM((C, E), jnp.float32),
            pltpu.VMEM((C, E), jnp.float32),
            pltpu.VMEM((C * K, E), jnp.float32),
            pltpu.VMEM((C,), jnp.float32),
            pltpu.VMEM((C * KPAD,), jnp.float32),
            pltpu.SemaphoreType.DMA,
        ]

    @functools.partial(
        pl.kernel,
        out_type=[
            jax.ShapeDtypeStruct((B,), jnp.float32),
            jax.ShapeDtypeStruct((B * KPAD,), jnp.float32),
        ],
        mesh=mesh,
        compiler_params=pltpu.CompilerParams(needs_layout_passes=False,
                                             use_tc_tiling_on_sc=False),
        scratch_types=one_set() + one_set(),
    )
    def sc_dots(center_hbm, context_hbm, negflat_hbm, inemb_hbm, outemb_hbm,
                pos_hbm, negdot_hbm, *scratch):
        set0, set1 = scratch[:9], scratch[9:]
        wid = lax.axis_index("s") * info.num_cores + lax.axis_index("c")
        wbase = wid * rows_per_w
        lane = lax.iota(jnp.int32, LANES)

        def remap(ref, n):
            for j in range(n // LANES):
                v = ref[pl.ds(j * LANES, LANES)]
                sel = (v >= H).astype(jnp.int32)
                ref[pl.ds(j * LANES, LANES)] = v * 2 - sel * (2 * H - 1)

        def gather_list(S):
            cidx, oidx, nidx, crow, orow, nrow, _, _, sem = S
            copies = [
                pltpu.make_async_copy(inemb_hbm.at[cidx], crow, sem),
                pltpu.make_async_copy(outemb_hbm.at[oidx], orow, sem),
            ]
            for j in range(C * K // IDX_BLK):
                copies.append(pltpu.make_async_copy(
                    outemb_hbm.at[nidx.at[pl.ds(j * IDX_BLK, IDX_BLK)]],
                    nrow.at[pl.ds(j * IDX_BLK, IDX_BLK)],
                    sem))
            return copies

        def stage_issue(g, S):
            cidx, oidx, nidx = S[0], S[1], S[2]
            sem = S[8]
            base = wbase + g * C
            stages = [
                pltpu.async_copy(center_hbm.at[pl.ds(base, C)], cidx, sem),
                pltpu.async_copy(context_hbm.at[pl.ds(base, C)], oidx, sem),
                pltpu.async_copy(negflat_hbm.at[pl.ds(base * K, C * K)],
                                 nidx, sem),
            ]
            for cp in stages:
                cp.wait()
            remap(cidx, C)
            remap(oidx, C)
            remap(nidx, C * K)
            for cp in gather_list(S):
                cp.start()

        def drain(S):
            for cp in gather_list(S):
                cp.wait()

        def compute_store(g, S):
            _, _, _, crow, orow, nrow, posv, negv, _ = S
            base = wbase + g * C

            def dot_rows(a_ref, a_row, b_ref, b_row):
                acc = (a_ref[a_row, pl.ds(0, LANES)]
                       * b_ref[b_row, pl.ds(0, LANES)])
                for v in range(1, EV):
                    acc = acc + (a_ref[a_row, pl.ds(v * LANES, LANES)]
                                 * b_ref[b_row, pl.ds(v * LANES, LANES)])
                return jnp.sum(acc, axis=0)

            def grp_body(grp, _):
                r0 = grp * LANES
                pvec = jnp.zeros((LANES,), jnp.float32)
                for i in range(LANES):
                    r = r0 + i
                    pvec = jnp.where(lane == i, dot_rows(crow, r, orow, r),
                                     pvec)
                    nvec0 = jnp.zeros((LANES,), jnp.float32)
                    nvec1 = jnp.zeros((LANES,), jnp.float32)
                    for k in range(K):
                        s = dot_rows(crow, r, nrow, r * K + k)
                        if k < LANES:
                            nvec0 = jnp.where(lane == k, s, nvec0)
                        else:
                            nvec1 = jnp.where(lane == (k - LANES), s, nvec1)
                    negv[pl.ds(r * KPAD, LANES)] = nvec0
                    negv[pl.ds(r * KPAD + LANES, LANES)] = nvec1
                posv[pl.ds(r0, LANES)] = pvec
                return 0

            lax.fori_loop(0, C // LANES, grp_body, 0)
            pltpu.sync_copy(posv, pos_hbm.at[pl.ds(base, C)])
            pltpu.sync_copy(negv, negdot_hbm.at[pl.ds(base * KPAD, C * KPAD)])

        stage_issue(0, set0)

        def pair_body(t, _):
            stage_issue(2 * t + 1, set1)
            drain(set0)
            compute_store(2 * t, set0)

            @pl.when(t < n_pairs - 1)
            def _():
                stage_issue(2 * t + 2, set0)

            drain(set1)
            compute_store(2 * t + 1, set1)
            return 0

        lax.fori_loop(0, n_pairs, pair_body, 0)

    return sc_dots


def _make_loss_body(B, K):
    def loss_body(pos_ref, neg_ref, out_ref):
        pos = pos_ref[...]
        neg = neg_ref[...]
        k_of_col = jax.lax.broadcasted_iota(jnp.int32, neg.shape, 1) % KPAD
        pos_term = -jnp.log(jax.nn.sigmoid(pos) + 1e-09)
        neg_term = jnp.where(k_of_col < K,
                             -jnp.log(jax.nn.sigmoid(-neg) + 1e-09), 0.0)
        out_ref[0, 0] = (jnp.sum(pos_term) + jnp.sum(neg_term)) / B
    return loss_body


def kernel(center, context, negatives, input_emb, output_emb):
    B, = center.shape
    K = negatives.shape[1]
    V, E = input_emb.shape

    relayout, H = _make_relayout(V, E)
    in_lin = relayout(input_emb)
    sc_relayout = _make_sc_relayout(V, E, H)
    out_lin = sc_relayout(
        output_emb.T.reshape(8, E // 8, V)).reshape(2 * H, E)

    sc_dots = _make_sc_dots(B, K, E, H)
    pos_dot, neg_dot = sc_dots(
        center.astype(jnp.int32),
        context.astype(jnp.int32),
        negatives.reshape(B * K).astype(jnp.int32),
        in_lin,
        out_lin,
    )

    loss = pl.pallas_call(
        _make_loss_body(B, K),
        out_shape=jax.ShapeDtypeStruct((1, 1), jnp.float32),
        in_specs=[
            pl.BlockSpec(memory_space=pltpu.VMEM),
            pl.BlockSpec(memory_space=pltpu.VMEM),
        ],
        out_specs=pl.BlockSpec(memory_space=pltpu.SMEM),
    )(pos_dot.reshape(B // 128, 128), neg_dot.reshape(B * KPAD // 128, 128))
    return loss.reshape(())


# speedup vs baseline: 3.3054x; 3.3054x over previous
import functools

import jax
import jax.numpy as jnp
from jax import lax
from jax.experimental import pallas as pl
from jax.experimental.pallas import tpu as pltpu
from jax.experimental.pallas import tpu_sc as plsc

LANES = 16
KPAD = 32
BI = 16384


def _make_relayout(V, E):
    NB = pl.cdiv(V, 2 * BI)
    H = NB * BI
    NBV = pl.cdiv(V, BI) - 1

    def body(up_ref, lo_ref, dst_ref):
        t_up = jnp.transpose(up_ref[...])
        t_lo = jnp.transpose(lo_ref[...])
        dst_ref[...] = jnp.concatenate([t_up, t_lo], axis=1)

    call = pl.pallas_call(
        body,
        grid=(NB,),
        in_specs=[
            pl.BlockSpec((E, BI), lambda i: (0, i)),
            pl.BlockSpec((E, BI), lambda i: (0, jnp.minimum(i + NB, NBV))),
        ],
        out_specs=pl.BlockSpec((BI, 2 * E), lambda i: (i, 0)),
        out_shape=jax.ShapeDtypeStruct((H, 2 * E), jnp.float32),
    )

    def relayout(table):
        tT = table.T
        return call(tT, tT).reshape(2 * H, E)

    return relayout, H


def _make_sc_dots(B, K, E, H):
    info = plsc.get_sparse_core_info()
    NW = info.num_cores * info.num_subcores
    rows_per_w = B // NW
    C = 32
    n_pairs = rows_per_w // (2 * C)
    EV = E // LANES
    IDX_BLK = 128

    mesh = plsc.VectorSubcoreMesh(core_axis_name="c", subcore_axis_name="s")

    def one_set():
        return [
            pltpu.VMEM((C,), jnp.int32),
            pltpu.VMEM((C,), jnp.int32),
            pltpu.VMEM((C * K,), jnp.int32),
            pltpu.VMEM((C, E), jnp.float32),
            pltpu.VMEM((C, E), jnp.float32),
            pltpu.VMEM((C * K, E), jnp.float32),
            pltpu.VMEM((C,), jnp.float32),
            pltpu.VMEM((C * KPAD,), jnp.float32),
            pltpu.SemaphoreType.DMA,
        ]

    @functools.partial(
        pl.kernel,
        out_type=[
            jax.ShapeDtypeStruct((B,), jnp.float32),
            jax.ShapeDtypeStruct((B * KPAD,), jnp.float32),
        ],
        mesh=mesh,
        compiler_params=pltpu.CompilerParams(needs_layout_passes=False,
                                             use_tc_tiling_on_sc=False),
        scratch_types=one_set() + one_set(),
    )
    def sc_dots(center_hbm, context_hbm, negflat_hbm, inemb_hbm, outemb_hbm,
                pos_hbm, negdot_hbm, *scratch):
        set0, set1 = scratch[:9], scratch[9:]
        wid = lax.axis_index("s") * info.num_cores + lax.axis_index("c")
        wbase = wid * rows_per_w
        lane = lax.iota(jnp.int32, LANES)

        def remap(ref, n):
            for j in range(n // LANES):
                v = ref[pl.ds(j * LANES, LANES)]
                sel = (v >= H).astype(jnp.int32)
                ref[pl.ds(j * LANES, LANES)] = v * 2 - sel * (2 * H - 1)

        def gather_list(S):
            cidx, oidx, nidx, crow, orow, nrow, _, _, sem = S
            copies = [
                pltpu.make_async_copy(inemb_hbm.at[cidx], crow, sem),
                pltpu.make_async_copy(outemb_hbm.at[oidx], orow, sem),
            ]
            for j in range(C * K // IDX_BLK):
                copies.append(pltpu.make_async_copy(
                    outemb_hbm.at[nidx.at[pl.ds(j * IDX_BLK, IDX_BLK)]],
                    nrow.at[pl.ds(j * IDX_BLK, IDX_BLK)],
                    sem))
            return copies

        def stage_issue(g, S):
            cidx, oidx, nidx = S[0], S[1], S[2]
            sem = S[8]
            base = wbase + g * C
            stages = [
                pltpu.async_copy(center_hbm.at[pl.ds(base, C)], cidx, sem),
                pltpu.async_copy(context_hbm.at[pl.ds(base, C)], oidx, sem),
                pltpu.async_copy(negflat_hbm.at[pl.ds(base * K, C * K)],
                                 nidx, sem),
            ]
            for cp in stages:
                cp.wait()
            remap(cidx, C)
            remap(oidx, C)
            remap(nidx, C * K)
            for cp in gather_list(S):
                cp.start()

        def drain(S):
            for cp in gather_list(S):
                cp.wait()

        def compute_store(g, S):
            _, _, _, crow, orow, nrow, posv, negv, _ = S
            base = wbase + g * C

            def dot_rows(a_ref, a_row, b_ref, b_row):
                acc = (a_ref[a_row, pl.ds(0, LANES)]
                       * b_ref[b_row, pl.ds(0, LANES)])
                for v in range(1, EV):
                    acc = acc + (a_ref[a_row, pl.ds(v * LANES, LANES)]
                                 * b_ref[b_row, pl.ds(v * LANES, LANES)])
                return jnp.sum(acc, axis=0)

            def grp_body(grp, _):
                r0 = grp * LANES
                pvec = jnp.zeros((LANES,), jnp.float32)
                for i in range(LANES):
                    r = r0 + i
                    pvec = jnp.where(lane == i, dot_rows(crow, r, orow, r),
                                     pvec)
                    nvec0 = jnp.zeros((LANES,), jnp.float32)
                    nvec1 = jnp.zeros((LANES,), jnp.float32)
                    for k in range(K):
                        s = dot_rows(crow, r, nrow, r * K + k)
                        if k < LANES:
                            nvec0 = jnp.where(lane == k, s, nvec0)
                        else:
                            nvec1 = jnp.where(lane == (k - LANES), s, nvec1)
                    negv[pl.ds(r * KPAD, LANES)] = nvec0
                    negv[pl.ds(r * KPAD + LANES, LANES)] = nvec1
                posv[pl.ds(r0, LANES)] = pvec
                return 0

            lax.fori_loop(0, C // LANES, grp_body, 0)
            pltpu.sync_copy(posv, pos_hbm.at[pl.ds(base, C)])
            pltpu.sync_copy(negv, negdot_hbm.at[pl.ds(base * KPAD, C * KPAD)])

        stage_issue(0, set0)

        def pair_body(t, _):
            stage_issue(2 * t + 1, set1)
            drain(set0)
            compute_store(2 * t, set0)

            @pl.when(t < n_pairs - 1)
            def _():
                stage_issue(2 * t + 2, set0)

            drain(set1)
            compute_store(2 * t + 1, set1)
            return 0

        lax.fori_loop(0, n_pairs, pair_body, 0)

    return sc_dots


def _make_loss_body(B, K):
    def loss_body(pos_ref, neg_ref, out_ref):
        pos = pos_ref[...]
        neg = neg_ref[...]
        k_of_col = jax.lax.broadcasted_iota(jnp.int32, neg.shape, 1) % KPAD
        pos_term = -jnp.log(jax.nn.sigmoid(pos) + 1e-09)
        neg_term = jnp.where(k_of_col < K,
                             -jnp.log(jax.nn.sigmoid(-neg) + 1e-09), 0.0)
        out_ref[0, 0] = (jnp.sum(pos_term) + jnp.sum(neg_term)) / B
    return loss_body


def kernel(center, context, negatives, input_emb, output_emb):
    B, = center.shape
    K = negatives.shape[1]
    V, E = input_emb.shape

    relayout, H = _make_relayout(V, E)
    in_lin = relayout(input_emb)
    out_lin = relayout(output_emb)

    sc_dots = _make_sc_dots(B, K, E, H)
    pos_dot, neg_dot = sc_dots(
        center.astype(jnp.int32),
        context.astype(jnp.int32),
        negatives.reshape(B * K).astype(jnp.int32),
        in_lin,
        out_lin,
    )

    loss = pl.pallas_call(
        _make_loss_body(B, K),
        out_shape=jax.ShapeDtypeStruct((1, 1), jnp.float32),
        in_specs=[
            pl.BlockSpec(memory_space=pltpu.VMEM),
            pl.BlockSpec(memory_space=pltpu.VMEM),
        ],
        out_specs=pl.BlockSpec(memory_space=pltpu.SMEM),
    )(pos_dot.reshape(B // 128, 128), neg_dot.reshape(B * KPAD // 128, 128))
    return loss.reshape(())
